# baseline (speedup 1.0000x reference)
import functools
import math

import jax
import jax.numpy as jnp
from jax import lax
from jax.experimental import pallas as pl
from jax.experimental.pallas import tpu as pltpu
from jax.experimental.pallas import tpu_sc as plsc

B = 2
S = 4096
HID = 2048
TOKEN_VOCAB = 10240
VOCAB = 16384
NUM_HEADS = 4
TOTAL_HEADS = 8
HEAD_DIM = 256
MEMORY_DIM = HEAD_DIM * TOTAL_HEADS
MOD = VOCAB - 1
CONV_K = 3

NC, NS = 2, 16
NW = NC * NS
C_TOK = S // 2
TPW = C_TOK // NW
CHUNK_T = 16
NHCHUNK = TPW // CHUNK_T
CHUNK_ROWS = TPW
IDS_PAD = 8

T_BLK = 512
NT = S // T_BLK


def _hash_constants():
    max_int = (1 << 31) - 1
    heads = []
    for n in (2, 3):
        for head_idx in range(NUM_HEADS):
            base_seed = 17 + 10007 * 1 + 1543 * (n + 1) + 8191 * (head_idx + 1)
            ms = []
            for pos in range(n):
                value = (base_seed + 32771 * (pos + 1)
                         + 65537 * (head_idx + 1) * (pos + 1)) % max_int
                m = value * 2 + 1
                ms.append((m & 0xFFFF, m >> 16))
            off = (base_seed * 2147483647 + 97 * (n + head_idx + 1)) % max_int
            heads.append((n, ms, (off & 0xFFFF, off >> 16)))
    return heads


_HASH_HEADS = _hash_constants()


def _c(v):
    return jnp.int32(v)


def _hash_one_head(toks, ms, olimbs):
    L0 = jnp.zeros((16,), jnp.int32)
    L1 = jnp.zeros((16,), jnp.int32)
    L2 = jnp.zeros((16,), jnp.int32)
    for t, (m0, m1) in zip(toks, ms):
        plo = t * _c(m0)
        phi = t * _c(m1)
        c1 = phi + (plo >> _c(16))
        L0 = L0 ^ (plo & _c(0xFFFF))
        L1 = L1 ^ (c1 & _c(0xFFFF))
        L2 = L2 ^ (c1 >> _c(16))
    o0, o1 = olimbs
    t0 = L0 + _c(o0)
    t1 = L1 + _c(o1) + (t0 >> _c(16))
    t2 = L2 + (t1 >> _c(16))
    y = (t0 & _c(0xFFFF)) + (t1 & _c(0xFFFF)) * _c(4) + t2 * _c(16)
    y = (y & _c(MOD)) + (y >> _c(14))
    y = (y & _c(MOD)) + (y >> _c(14))
    y = jnp.where(y >= _c(MOD), y - _c(MOD), y)
    return y + _c(1)


def _sc_body(chunk_off, ids_hbm, table_hbm, out_hbm, ids_v, idx_v, rows_a,
             rows_b, sem_a, sem_b):
    wid = (lax.axis_index("s") * _c(NC) + lax.axis_index("c")).astype(jnp.int32)
    base_local = wid * _c(TPW)

    pltpu.sync_copy(ids_hbm.at[pl.ds(base_local, TPW + IDS_PAD)], ids_v)

    lane = lax.broadcasted_iota(jnp.int32, (16,), 0)

    def hash_chunk(k, _):
        off = _c(IDS_PAD) + k * _c(CHUNK_T)
        tok0 = ids_v[pl.ds(off, CHUNK_T)]
        tok1 = ids_v[pl.ds(off - _c(1), CHUNK_T)]
        tok2 = ids_v[pl.ds(off - _c(2), CHUNK_T)]
        tl = _c(chunk_off) + base_local + k * _c(CHUNK_T) + lane
        col = k * _c(CHUNK_T)
        for h, (n, ms, olimbs) in enumerate(_HASH_HEADS):
            toks = [tok1, tok0] if n == 2 else [tok2, tok1, tok0]
            hashed = _hash_one_head(toks, ms, olimbs)
            hashed = jnp.where(tl >= _c(n - 1), hashed, _c(0))
            idx_v[_c(h), pl.ds(col, CHUNK_T)] = hashed + _c(h * VOCAB)
        return 0

    lax.fori_loop(jnp.int32(0), jnp.int32(NHCHUNK), hash_chunk, 0)

    bufs = (rows_a, rows_b)
    sems = (sem_a, sem_b)
    copies = [None, None]

    def start_gather(h, buf, sem):
        return pltpu.async_copy(table_hbm.at[idx_v.at[_c(h)]], buf, sem)

    copies[0] = start_gather(0, bufs[0], sems[0])
    for h in range(TOTAL_HEADS):
        cur = h % 2
        if h + 1 < TOTAL_HEADS:
            nxt = (h + 1) % 2
            copies[nxt] = start_gather(h + 1, bufs[nxt], sems[nxt])
        copies[cur].wait()
        pltpu.sync_copy(bufs[cur],
                        out_hbm.at[_c(h), pl.ds(base_local, CHUNK_ROWS)])


@functools.cache
def _sc_gather_fn(chunk_off):
    return functools.partial(
        pl.kernel,
        mesh=plsc.VectorSubcoreMesh(core_axis_name="c", subcore_axis_name="s"),
        out_type=jax.ShapeDtypeStruct((TOTAL_HEADS, C_TOK, HEAD_DIM),
                                      jnp.float32),
        scratch_types=[
            pltpu.VMEM((TPW + IDS_PAD,), jnp.int32),
            pltpu.VMEM((TOTAL_HEADS, TPW), jnp.int32),
            pltpu.VMEM((CHUNK_ROWS, HEAD_DIM), jnp.float32),
            pltpu.VMEM((CHUNK_ROWS, HEAD_DIM), jnp.float32),
            pltpu.SemaphoreType.DMA,
            pltpu.SemaphoreType.DMA,
        ],
    )(functools.partial(_sc_body, chunk_off))


def _tc_core(mem_ref, hid_ref, wk_ref, wv_ref, aux_ref, tail_ref, out_ref,
             tout_ref, scr_ref):
    j = pl.program_id(0)
    hid = hid_ref[0]
    kk = None
    vv = None
    dn = (((1,), (1,)), ((), ()))
    for h in range(TOTAL_HEADS):
        mh = mem_ref[h].astype(jnp.bfloat16)
        pk = lax.dot_general(mh, wk_ref[:, pl.ds(h * HEAD_DIM, HEAD_DIM)],
                             dn, precision=lax.Precision.DEFAULT,
                             preferred_element_type=jnp.float32)
        pv = lax.dot_general(mh, wv_ref[:, pl.ds(h * HEAD_DIM, HEAD_DIM)],
                             dn, precision=lax.Precision.DEFAULT,
                             preferred_element_type=jnp.float32)
        kk = pk if kk is None else kk + pk
        vv = pv if vv is None else vv + pv
    kvar = jnp.mean(kk * kk, axis=-1, keepdims=True)
    mk = kk * lax.rsqrt(kvar + 1e-6) * aux_ref[0:1, :]
    g = jax.nn.sigmoid(jnp.sum(hid * mk, axis=-1, keepdims=True)
                       * (1.0 / math.sqrt(HID)))
    vvar = jnp.mean(vv * vv, axis=-1, keepdims=True)
    mv = vv * lax.rsqrt(vvar + 1e-6) * aux_ref[1:2, :]
    gated = g * mv

    @pl.when(j == 0)
    def _():
        scr_ref[pl.ds(6, 2), :] = tail_ref[...]

    scr_ref[pl.ds(8, T_BLK), :] = gated
    conv = (scr_ref[pl.ds(6, T_BLK), :] * aux_ref[2:3, :]
            + scr_ref[pl.ds(7, T_BLK), :] * aux_ref[3:4, :]
            + gated * aux_ref[4:5, :])
    out_ref[0] = hid + conv

    @pl.when(j == NTC - 1)
    def _():
        tout_ref[...] = scr_ref[pl.ds(T_BLK + 6, 2), :]

    scr_ref[pl.ds(6, 2), :] = scr_ref[pl.ds(T_BLK + 6, 2), :]


def _z():
    return jnp.int32(0)


def _tc_body_noalias(mem_ref, hid_ref, wk_ref, wv_ref, aux_ref, tail_ref,
                     out_ref, tout_ref, scr_ref):
    _tc_core(mem_ref, hid_ref, wk_ref, wv_ref, aux_ref, tail_ref, out_ref,
             tout_ref, scr_ref)


def _tc_body_alias(mem_ref, hid_ref, wk_ref, wv_ref, aux_ref, tail_ref,
                   prev_ref, out_ref, tout_ref, scr_ref):
    del prev_ref
    _tc_core(mem_ref, hid_ref, wk_ref, wv_ref, aux_ref, tail_ref, out_ref,
             tout_ref, scr_ref)


NTC = C_TOK // T_BLK


def _tc_fused(mem3, hidden_states, Wkb, Wvb, aux, tail_in, batch, half,
              prev=None):
    blk0 = half * NTC
    in_specs = [
        pl.BlockSpec((TOTAL_HEADS, T_BLK, HEAD_DIM),
                     lambda j: (_z(), j, _z())),
        pl.BlockSpec((1, T_BLK, HID), lambda j: (_c(batch), _c(blk0) + j,
                                                 _z())),
        pl.BlockSpec((HID, MEMORY_DIM), lambda j: (_z(), _z())),
        pl.BlockSpec((HID, MEMORY_DIM), lambda j: (_z(), _z())),
        pl.BlockSpec((8, HID), lambda j: (_z(), _z())),
        pl.BlockSpec((2, HID), lambda j: (_z(), _z())),
    ]
    args = [mem3, hidden_states, Wkb, Wvb, aux, tail_in]
    kwargs = {}
    body = _tc_body_noalias
    if prev is not None:
        in_specs.append(pl.BlockSpec(memory_space=pl.ANY))
        args.append(prev)
        kwargs["input_output_aliases"] = {6: 0}
        body = _tc_body_alias
    return pl.pallas_call(
        body,
        grid=(NTC,),
        in_specs=in_specs,
        out_specs=[
            pl.BlockSpec((1, T_BLK, HID), lambda j: (_c(batch), _c(blk0) + j,
                                                     _z())),
            pl.BlockSpec((2, HID), lambda j: (_z(), _z())),
        ],
        out_shape=[
            jax.ShapeDtypeStruct((B, S, HID), jnp.float32),
            jax.ShapeDtypeStruct((2, HID), jnp.float32),
        ],
        scratch_shapes=[pltpu.VMEM((T_BLK + 8, HID), jnp.float32)],
        compiler_params=pltpu.CompilerParams(
            dimension_semantics=("arbitrary",)),
        **kwargs,
    )(*args)


def kernel(hidden_states, input_ids, tables, Wk, Wv, key_norm_w, value_norm_w,
           conv_w):
    ids32 = input_ids.astype(jnp.int32)
    ids_pad = jnp.pad(ids32, ((0, 0), (IDS_PAD, 0)))
    table_flat = tables.reshape(TOTAL_HEADS * VOCAB, HEAD_DIM)

    aux = jnp.zeros((8, HID), jnp.float32)
    aux = aux.at[0].set(key_norm_w)
    aux = aux.at[1].set(value_norm_w)
    aux = aux.at[2:5].set(conv_w.T)
    Wkb = Wk.astype(jnp.bfloat16)
    Wvb = Wv.astype(jnp.bfloat16)

    zero_tail = jnp.zeros((2, HID), jnp.float32)
    out = None
    tail = zero_tail
    for chunk in range(2 * B):
        batch, half = chunk // 2, chunk % 2
        ids_c = lax.dynamic_slice(ids_pad[batch], (half * C_TOK,),
                                  (C_TOK + IDS_PAD,))
        mem3 = _sc_gather_fn(half * C_TOK)(ids_c, table_flat)
        tail_in = zero_tail if half == 0 else tail
        out, tail = _tc_fused(mem3, hidden_states, Wkb, Wvb, aux, tail_in,
                              batch, half, prev=out)
    return out


# speedup vs baseline: 1.1156x; 1.1156x over previous
import functools
import math

import jax
import jax.numpy as jnp
from jax import lax
from jax.experimental import pallas as pl
from jax.experimental.pallas import tpu as pltpu
from jax.experimental.pallas import tpu_sc as plsc

B = 2
S = 4096
HID = 2048
TOKEN_VOCAB = 10240
VOCAB = 16384
NUM_HEADS = 4
TOTAL_HEADS = 8
HEAD_DIM = 256
MEMORY_DIM = HEAD_DIM * TOTAL_HEADS
MOD = VOCAB - 1
CONV_K = 3

NC, NS = 2, 16
NW = NC * NS
TPW = S // NW
CHUNK_T = 16
NHCHUNK = TPW // CHUNK_T
CHUNK_ROWS = TPW
IDS_PAD = 8

T_BLK = 512
NT = S // T_BLK


def _hash_constants():
    max_int = (1 << 31) - 1
    heads = []
    for n in (2, 3):
        for head_idx in range(NUM_HEADS):
            base_seed = 17 + 10007 * 1 + 1543 * (n + 1) + 8191 * (head_idx + 1)
            ms = []
            for pos in range(n):
                value = (base_seed + 32771 * (pos + 1)
                         + 65537 * (head_idx + 1) * (pos + 1)) % max_int
                m = value * 2 + 1
                ms.append((m & 0xFFFF, m >> 16))
            off = (base_seed * 2147483647 + 97 * (n + head_idx + 1)) % max_int
            heads.append((n, ms, (off & 0xFFFF, off >> 16)))
    return heads


_HASH_HEADS = _hash_constants()


def _c(v):
    return jnp.int32(v)


def _hash_one_head(toks, ms, olimbs):
    L0 = jnp.zeros((16,), jnp.int32)
    L1 = jnp.zeros((16,), jnp.int32)
    L2 = jnp.zeros((16,), jnp.int32)
    for t, (m0, m1) in zip(toks, ms):
        plo = t * _c(m0)
        phi = t * _c(m1)
        c1 = phi + (plo >> _c(16))
        L0 = L0 ^ (plo & _c(0xFFFF))
        L1 = L1 ^ (c1 & _c(0xFFFF))
        L2 = L2 ^ (c1 >> _c(16))
    o0, o1 = olimbs
    t0 = L0 + _c(o0)
    t1 = L1 + _c(o1) + (t0 >> _c(16))
    t2 = L2 + (t1 >> _c(16))
    y = (t0 & _c(0xFFFF)) + (t1 & _c(0xFFFF)) * _c(4) + t2 * _c(16)
    y = (y & _c(MOD)) + (y >> _c(14))
    y = (y & _c(MOD)) + (y >> _c(14))
    y = jnp.where(y >= _c(MOD), y - _c(MOD), y)
    return y + _c(1)


def _sc_body(ids_hbm, table_hbm, out_hbm, ids_v, idx_v, rows_a, rows_b,
             sem_a, sem_b):
    wid = (lax.axis_index("s") * _c(NC) + lax.axis_index("c")).astype(jnp.int32)
    base_local = wid * _c(TPW)

    pltpu.sync_copy(ids_hbm.at[pl.ds(base_local, TPW + IDS_PAD)], ids_v)

    lane = lax.broadcasted_iota(jnp.int32, (16,), 0)

    def hash_chunk(k, _):
        off = _c(IDS_PAD) + k * _c(CHUNK_T)
        tok0 = ids_v[pl.ds(off, CHUNK_T)]
        tok1 = ids_v[pl.ds(off - _c(1), CHUNK_T)]
        tok2 = ids_v[pl.ds(off - _c(2), CHUNK_T)]
        tl = base_local + k * _c(CHUNK_T) + lane
        col = k * _c(CHUNK_T)
        for h, (n, ms, olimbs) in enumerate(_HASH_HEADS):
            toks = [tok1, tok0] if n == 2 else [tok2, tok1, tok0]
            hashed = _hash_one_head(toks, ms, olimbs)
            hashed = jnp.where(tl >= _c(n - 1), hashed, _c(0))
            idx_v[_c(h), pl.ds(col, CHUNK_T)] = hashed + _c(h * VOCAB)
        return 0

    lax.fori_loop(jnp.int32(0), jnp.int32(NHCHUNK), hash_chunk, 0)

    bufs = (rows_a, rows_b)
    sems = (sem_a, sem_b)
    copies = [None, None]

    def start_gather(h, buf, sem):
        return pltpu.async_copy(table_hbm.at[idx_v.at[_c(h)]], buf, sem)

    copies[0] = start_gather(0, bufs[0], sems[0])
    for h in range(TOTAL_HEADS):
        cur = h % 2
        if h + 1 < TOTAL_HEADS:
            nxt = (h + 1) % 2
            copies[nxt] = start_gather(h + 1, bufs[nxt], sems[nxt])
        copies[cur].wait()
        pltpu.sync_copy(bufs[cur],
                        out_hbm.at[_c(h), pl.ds(base_local, CHUNK_ROWS)])


@functools.cache
def _sc_gather_fn():
    return functools.partial(
        pl.kernel,
        mesh=plsc.VectorSubcoreMesh(core_axis_name="c", subcore_axis_name="s"),
        out_type=jax.ShapeDtypeStruct((TOTAL_HEADS, S, HEAD_DIM),
                                      jnp.float32),
        scratch_types=[
            pltpu.VMEM((TPW + IDS_PAD,), jnp.int32),
            pltpu.VMEM((TOTAL_HEADS, TPW), jnp.int32),
            pltpu.VMEM((CHUNK_ROWS, HEAD_DIM), jnp.float32),
            pltpu.VMEM((CHUNK_ROWS, HEAD_DIM), jnp.float32),
            pltpu.SemaphoreType.DMA,
            pltpu.SemaphoreType.DMA,
        ],
    )(_sc_body)


def _tc_core(mem_ref, hid_ref, wk_ref, wv_ref, aux_ref, out_ref, scr_ref):
    j = pl.program_id(0)
    hid = hid_ref[0]
    kk = None
    vv = None
    dn = (((1,), (1,)), ((), ()))
    for h in range(TOTAL_HEADS):
        mh = mem_ref[h].astype(jnp.bfloat16)
        pk = lax.dot_general(mh, wk_ref[:, pl.ds(h * HEAD_DIM, HEAD_DIM)],
                             dn, precision=lax.Precision.DEFAULT,
                             preferred_element_type=jnp.float32)
        pv = lax.dot_general(mh, wv_ref[:, pl.ds(h * HEAD_DIM, HEAD_DIM)],
                             dn, precision=lax.Precision.DEFAULT,
                             preferred_element_type=jnp.float32)
        kk = pk if kk is None else kk + pk
        vv = pv if vv is None else vv + pv
    kvar = jnp.mean(kk * kk, axis=-1, keepdims=True)
    mk = kk * lax.rsqrt(kvar + 1e-6) * aux_ref[0:1, :]
    g = jax.nn.sigmoid(jnp.sum(hid * mk, axis=-1, keepdims=True)
                       * (1.0 / math.sqrt(HID)))
    vvar = jnp.mean(vv * vv, axis=-1, keepdims=True)
    mv = vv * lax.rsqrt(vvar + 1e-6) * aux_ref[1:2, :]
    gated = g * mv

    @pl.when(j == 0)
    def _():
        scr_ref[pl.ds(6, 2), :] = jnp.zeros((2, HID), jnp.float32)

    scr_ref[pl.ds(8, T_BLK), :] = gated
    conv = (scr_ref[pl.ds(6, T_BLK), :] * aux_ref[2:3, :]
            + scr_ref[pl.ds(7, T_BLK), :] * aux_ref[3:4, :]
            + gated * aux_ref[4:5, :])
    out_ref[0] = hid + conv
    scr_ref[pl.ds(6, 2), :] = scr_ref[pl.ds(T_BLK + 6, 2), :]


def _z():
    return jnp.int32(0)


def _tc_body_noalias(mem_ref, hid_ref, wk_ref, wv_ref, aux_ref, out_ref,
                     scr_ref):
    _tc_core(mem_ref, hid_ref, wk_ref, wv_ref, aux_ref, out_ref, scr_ref)


def _tc_body_alias(mem_ref, hid_ref, wk_ref, wv_ref, aux_ref, prev_ref,
                   out_ref, scr_ref):
    del prev_ref
    _tc_core(mem_ref, hid_ref, wk_ref, wv_ref, aux_ref, out_ref, scr_ref)


def _tc_fused(mem3, hidden_states, Wkb, Wvb, aux, batch, prev=None):
    in_specs = [
        pl.BlockSpec((TOTAL_HEADS, T_BLK, HEAD_DIM),
                     lambda j: (_z(), j, _z())),
        pl.BlockSpec((1, T_BLK, HID), lambda j: (_c(batch), j, _z())),
        pl.BlockSpec((HID, MEMORY_DIM), lambda j: (_z(), _z())),
        pl.BlockSpec((HID, MEMORY_DIM), lambda j: (_z(), _z())),
        pl.BlockSpec((8, HID), lambda j: (_z(), _z())),
    ]
    args = [mem3, hidden_states, Wkb, Wvb, aux]
    kwargs = {}
    body = _tc_body_noalias
    if prev is not None:
        in_specs.append(pl.BlockSpec(memory_space=pl.ANY))
        args.append(prev)
        kwargs["input_output_aliases"] = {5: 0}
        body = _tc_body_alias
    return pl.pallas_call(
        body,
        grid=(NT,),
        in_specs=in_specs,
        out_specs=pl.BlockSpec((1, T_BLK, HID), lambda j: (_c(batch), j, _z())),
        out_shape=jax.ShapeDtypeStruct((B, S, HID), jnp.float32),
        scratch_shapes=[pltpu.VMEM((T_BLK + 8, HID), jnp.float32)],
        compiler_params=pltpu.CompilerParams(
            dimension_semantics=("arbitrary",)),
        **kwargs,
    )(*args)


def kernel(hidden_states, input_ids, tables, Wk, Wv, key_norm_w, value_norm_w,
           conv_w):
    ids32 = input_ids.astype(jnp.int32)
    ids_pad = jnp.pad(ids32, ((0, 0), (IDS_PAD, 0)))
    table_flat = tables.reshape(TOTAL_HEADS * VOCAB, HEAD_DIM)

    aux = jnp.zeros((8, HID), jnp.float32)
    aux = aux.at[0].set(key_norm_w)
    aux = aux.at[1].set(value_norm_w)
    aux = aux.at[2:5].set(conv_w.T)
    Wkb = Wk.astype(jnp.bfloat16)
    Wvb = Wv.astype(jnp.bfloat16)

    sc = _sc_gather_fn()
    out = None
    for batch in range(B):
        mem3 = sc(ids_pad[batch], table_flat)
        out = _tc_fused(mem3, hidden_states, Wkb, Wvb, aux, batch, prev=out)
    return out
